# Initial kernel scaffold; baseline (speedup 1.0000x reference)
import jax
import jax.numpy as jnp
from jax.experimental import pallas as pl


def kernel(x, router_w, w1, w2, w3):
    raise NotImplementedError("write your pallas kernel here")


# speedup vs baseline: 2.2413x; 2.2413x over previous
import functools

import jax
import jax.numpy as jnp
from jax.experimental import pallas as pl
from jax.experimental.pallas import tpu as pltpu

DIM = 768
NUM_EXPERTS = 8
TOP_K = 2
HID = int(DIM * 1.5)
N_TOKENS = 2048
LANES = 128


def _router_body(x_ref, rw_ref, combine_ref):
    x = x_ref[...]
    rw = rw_ref[...]
    logits = jax.lax.dot_general(
        x, rw, (((1,), (1,)), ((), ())), preferred_element_type=jnp.float32)
    lane = jax.lax.broadcasted_iota(jnp.int32, logits.shape, 1)
    neg_inf = jnp.float32(-jnp.inf)
    logits = jnp.where(lane < NUM_EXPERTS, logits, neg_inf)
    m1 = jnp.max(logits, axis=1, keepdims=True)
    is1 = logits == m1
    a1 = jnp.min(jnp.where(is1, lane, LANES), axis=1, keepdims=True)
    masked = jnp.where(lane == a1, neg_inf, logits)
    m2 = jnp.max(masked, axis=1, keepdims=True)
    is2 = masked == m2
    a2 = jnp.min(jnp.where(is2, lane, LANES), axis=1, keepdims=True)
    p1 = jax.nn.sigmoid(m1 - m2)
    p2 = 1.0 - p1
    combine = jnp.where(lane == a1, p1, 0.0) + jnp.where(lane == a2, p2, 0.0)
    combine_ref[...] = combine


def _expert_body(x_ref, w1_ref, w2_ref, w3_ref, c_ref, out_ref):
    e = pl.program_id(0)
    x = x_ref[...]
    w1 = w1_ref[0]
    w3 = w3_ref[0]
    w2 = w2_ref[0]
    g = jax.lax.dot_general(
        x, w1, (((1,), (1,)), ((), ())), preferred_element_type=jnp.float32)
    u = jax.lax.dot_general(
        x, w3, (((1,), (1,)), ((), ())), preferred_element_type=jnp.float32)
    h = g * jax.nn.sigmoid(g) * u
    y = jax.lax.dot_general(
        h, w2, (((1,), (1,)), ((), ())), preferred_element_type=jnp.float32)
    lane = jax.lax.broadcasted_iota(jnp.int32, c_ref.shape, 1)
    ce = jnp.sum(jnp.where(lane == e, c_ref[...], 0.0), axis=1, keepdims=True)
    contrib = y * ce

    @pl.when(e == 0)
    def _():
        out_ref[...] = contrib

    @pl.when(e != 0)
    def _():
        out_ref[...] += contrib


def kernel(x, router_w, w1, w2, w3):
    B, T, C = x.shape
    x_flat = x.reshape(-1, C)
    n = x_flat.shape[0]
    rw_pad = jnp.zeros((LANES, C), x.dtype).at[:NUM_EXPERTS].set(router_w)

    combine = pl.pallas_call(
        _router_body,
        out_shape=jax.ShapeDtypeStruct((n, LANES), jnp.float32),
        in_specs=[
            pl.BlockSpec((n, C), lambda: (0, 0)),
            pl.BlockSpec((LANES, C), lambda: (0, 0)),
        ],
        out_specs=pl.BlockSpec((n, LANES), lambda: (0, 0)),
    )(x_flat, rw_pad)

    out = pl.pallas_call(
        _expert_body,
        grid=(NUM_EXPERTS,),
        out_shape=jax.ShapeDtypeStruct((n, C), jnp.float32),
        in_specs=[
            pl.BlockSpec((n, C), lambda e: (0, 0)),
            pl.BlockSpec((1, HID, C), lambda e: (e, 0, 0)),
            pl.BlockSpec((1, C, HID), lambda e: (e, 0, 0)),
            pl.BlockSpec((1, HID, C), lambda e: (e, 0, 0)),
            pl.BlockSpec((n, LANES), lambda e: (0, 0)),
        ],
        out_specs=pl.BlockSpec((n, C), lambda e: (0, 0)),
    )(x_flat, w1, w2, w3, combine)
    return out.reshape(B, T, C)
